# baseline (speedup 1.0000x reference)
import functools

import jax
import jax.numpy as jnp
from jax import lax
from jax.experimental import pallas as pl
from jax.experimental.pallas import tpu as pltpu
from jax.experimental.pallas import tpu_sc as plsc

_F = 26
_D = 16
_B = 16384
_FS = 100000
_V = _F * _FS
_L = 16
_BSC = _B // 2
_PLANE = 100224


def _chunk(f):
    lo = f * _FS
    hi = min((f + 1) * _FS, _V)
    r0 = (lo // 128) * 128
    r1 = min(((hi + 127) // 128) * 128, (_V // 128) * 128)
    ln = r1 - r0
    tail_dst = ln if hi > r1 else -1
    lnA = ((ln // 2) // 128) * 128
    return r0, ln, lnA, lo - r0, tail_dst


def _fm_body(xt_hbm, embT_hbm, lin_hbm, embtail_hbm, lintail_hbm, out_hbm,
             plane, s_acc, m_acc, idxbuf, outbuf, ssh, semA, semB):
    cid = lax.axis_index("c")
    sid = lax.axis_index("s")
    b0 = cid * _BSC
    zero = jnp.zeros((_L,), jnp.float32)

    def issue_a(f):
        r0, ln, lnA, off, tail = _chunk(f)
        return [pltpu.async_copy(embT_hbm.at[sid, pl.ds(r0, lnA)],
                                 plane.at[pl.ds(0, lnA)], semA)]

    def issue_b(f):
        r0, ln, lnA, off, tail = _chunk(f)
        cps = [pltpu.async_copy(embT_hbm.at[sid, pl.ds(r0 + lnA, ln - lnA)],
                                plane.at[pl.ds(lnA, ln - lnA)], semB)]
        if tail >= 0:
            cps.append(pltpu.async_copy(embtail_hbm.at[sid],
                                        plane.at[pl.ds(tail, 128)], semB))
        return cps

    cps_a = issue_a(0)
    cps_b = issue_b(0)

    def zero_body(j, c):
        s_acc[pl.ds(j * _L, _L)] = zero
        m_acc[pl.ds(j * _L, _L)] = zero
        return c

    lax.fori_loop(0, _BSC // _L, zero_body, 0)

    def gather_pass(off, lnA, half):
        def j_body(j, c):
            for u in range(4):
                sl = pl.ds(j * 4 * _L + u * _L, _L)
                i16 = idxbuf[sl] + off
                msk = (i16 < lnA) if half == 0 else (i16 >= lnA)
                v = plsc.load_gather(plane, [i16])
                v = jnp.where(msk, v, 0.0)
                plsc.addupdate(s_acc.at[sl], v)
                plsc.addupdate(m_acc.at[sl], v * v)
            return c

        lax.fori_loop(0, _BSC // (4 * _L), j_body, 0)

    for f in range(_F):
        r0, ln, lnA, off, tail = _chunk(f)
        pltpu.sync_copy(xt_hbm.at[f, pl.ds(b0, _BSC)], idxbuf)
        for cp in cps_a:
            cp.wait()
        gather_pass(off, lnA, 0)
        if f + 1 < _F:
            cps_a = issue_a(f + 1)
        for cp in cps_b:
            cp.wait()
        gather_pass(off, lnA, 1)
        if f + 1 < _F:
            cps_b = issue_b(f + 1)

    sub = sid * (_BSC // 16)
    nsub = _BSC // 16
    xbase = cid * 16 * _BSC
    s_pub = pltpu.async_copy(s_acc, ssh.at[pl.ds(xbase + sid * _BSC, _BSC)],
                             semA)

    for f in range(_F):
        r0, ln, lnA, off, tail = _chunk(f)

        @pl.when(sid == (f % 16))
        def _do_lin(f=f, r0=r0, ln=ln, off=off, tail=tail):
            pltpu.sync_copy(lin_hbm.at[0, pl.ds(r0, ln)], plane.at[pl.ds(0, ln)])
            if tail >= 0:
                pltpu.sync_copy(lintail_hbm, plane.at[pl.ds(tail, 128)])
            pltpu.sync_copy(xt_hbm.at[f, pl.ds(b0, _BSC)], idxbuf)

            def j_body(j, c):
                i16 = idxbuf[pl.ds(j * _L, _L)] + off
                w = plsc.load_gather(plane, [i16])
                plsc.addupdate(m_acc.at[pl.ds(j * _L, _L)], -(w + w))
                return c

            lax.fori_loop(0, _BSC // _L, j_body, 0)

    s_pub.wait()
    plsc.subcore_barrier()
    for d in range(_D):
        pltpu.sync_copy(ssh.at[pl.ds(xbase + d * _BSC + sub, nsub)],
                        s_acc.at[pl.ds(d * nsub, nsub)])

    def s_body(g, c):
        acc = zero
        for d in range(_D):
            sv = s_acc[pl.ds(d * nsub + g * _L, _L)]
            acc = acc + sv * sv
        outbuf[pl.ds(g * _L, _L)] = acc
        return c

    lax.fori_loop(0, nsub // _L, s_body, 0)

    plsc.subcore_barrier()
    pltpu.sync_copy(m_acc, ssh.at[pl.ds(xbase + sid * _BSC, _BSC)])
    plsc.subcore_barrier()
    for d in range(_D):
        pltpu.sync_copy(ssh.at[pl.ds(xbase + d * _BSC + sub, nsub)],
                        s_acc.at[pl.ds(d * nsub, nsub)])

    def m_body(g, c):
        mtot = zero
        for d in range(_D):
            mtot = mtot + s_acc[pl.ds(d * nsub + g * _L, _L)]
        sl = pl.ds(g * _L, _L)
        outbuf[sl] = 0.5 * (outbuf[sl] - mtot)
        return c

    lax.fori_loop(0, nsub // _L, m_body, 0)
    pltpu.sync_copy(outbuf, out_hbm.at[pl.ds(b0 + sub, nsub)])


@jax.jit
def _fm(xt, embT, lin_flat, emb_tail, lin_tail):
    run = functools.partial(
        pl.kernel,
        out_type=jax.ShapeDtypeStruct((_B,), jnp.float32),
        mesh=plsc.VectorSubcoreMesh(core_axis_name="c", subcore_axis_name="s"),
        compiler_params=pltpu.CompilerParams(
            use_tc_tiling_on_sc=True, needs_layout_passes=False,
            disable_bounds_checks=True),
        scratch_types=[
            pltpu.VMEM((_PLANE,), jnp.float32),
            pltpu.VMEM((_BSC,), jnp.float32),
            pltpu.VMEM((_BSC,), jnp.float32),
            pltpu.VMEM((_BSC,), jnp.int32),
            pltpu.VMEM((_BSC // 16,), jnp.float32),
            pltpu.HBM((2 * 16 * _BSC,), jnp.float32),
            pltpu.SemaphoreType.DMA,
            pltpu.SemaphoreType.DMA,
        ],
    )(_fm_body)
    return run(xt, embT, lin_flat, emb_tail, lin_tail)


def kernel(x, emb_table, lin_weight, lin_bias):
    emb_tail = jnp.pad(emb_table[_V - 64:].T, ((0, 0), (0, 64)))
    lin_tail = jnp.pad(lin_weight[_V - 64:, 0], (0, 64))
    out = _fm(x.T, emb_table.T, lin_weight.T, emb_tail, lin_tail)
    return out[:, None] + lin_bias[None, :]


# speedup vs baseline: 1.0089x; 1.0089x over previous
import functools

import jax
import jax.numpy as jnp
from jax import lax
from jax.experimental import pallas as pl
from jax.experimental.pallas import tpu as pltpu
from jax.experimental.pallas import tpu_sc as plsc

_F = 26
_D = 16
_B = 16384
_FS = 100000
_V = _F * _FS
_L = 16
_BSC = _B // 2
_PLANE = 100224


def _chunk(f):
    lo = f * _FS
    hi = min((f + 1) * _FS, _V)
    r0 = (lo // 128) * 128
    r1 = min(((hi + 127) // 128) * 128, (_V // 128) * 128)
    ln = r1 - r0
    tail_dst = ln if hi > r1 else -1
    lnA = ((ln // 2) // 128) * 128
    return r0, ln, lnA, lo - r0, tail_dst


def _fm_body(xt_hbm, embT_hbm, lin_hbm, embtail_hbm, lintail_hbm, out_hbm,
             plane, s_acc, m_acc, idxbuf, outbuf, ssh, semA, semB):
    cid = lax.axis_index("c")
    sid = lax.axis_index("s")
    b0 = cid * _BSC
    zero = jnp.zeros((_L,), jnp.float32)

    def issue_a(f):
        r0, ln, lnA, off, tail = _chunk(f)
        return [pltpu.async_copy(embT_hbm.at[sid, pl.ds(r0, lnA)],
                                 plane.at[pl.ds(0, lnA)], semA)]

    def issue_b(f):
        r0, ln, lnA, off, tail = _chunk(f)
        cps = [pltpu.async_copy(embT_hbm.at[sid, pl.ds(r0 + lnA, ln - lnA)],
                                plane.at[pl.ds(lnA, ln - lnA)], semB)]
        if tail >= 0:
            cps.append(pltpu.async_copy(embtail_hbm.at[sid],
                                        plane.at[pl.ds(tail, 128)], semB))
        return cps

    cps_a = issue_a(0)
    cps_b = issue_b(0)

    def zero_body(j, c):
        s_acc[pl.ds(j * _L, _L)] = zero
        m_acc[pl.ds(j * _L, _L)] = zero
        return c

    lax.fori_loop(0, _BSC // _L, zero_body, 0)

    def gather_pass(off, lnA, half):
        def j_body(j, c):
            for u in range(2):
                sl = pl.ds(j * 2 * _L + u * _L, _L)
                i16 = idxbuf[sl] + off
                msk = (i16 < lnA) if half == 0 else (i16 >= lnA)
                v = plsc.load_gather(plane, [i16])
                v = jnp.where(msk, v, 0.0)
                plsc.addupdate(s_acc.at[sl], v)
                plsc.addupdate(m_acc.at[sl], v * v)
            return c

        lax.fori_loop(0, _BSC // (2 * _L), j_body, 0)

    for f in range(_F):
        r0, ln, lnA, off, tail = _chunk(f)
        pltpu.sync_copy(xt_hbm.at[f, pl.ds(b0, _BSC)], idxbuf)
        for cp in cps_a:
            cp.wait()
        gather_pass(off, lnA, 0)
        if f + 1 < _F:
            cps_a = issue_a(f + 1)
        for cp in cps_b:
            cp.wait()
        gather_pass(off, lnA, 1)
        if f + 1 < _F:
            cps_b = issue_b(f + 1)

    sub = sid * (_BSC // 16)
    nsub = _BSC // 16
    xbase = cid * 16 * _BSC
    s_pub = pltpu.async_copy(s_acc, ssh.at[pl.ds(xbase + sid * _BSC, _BSC)],
                             semA)

    for f in range(_F):
        r0, ln, lnA, off, tail = _chunk(f)

        @pl.when(sid == (f % 16))
        def _do_lin(f=f, r0=r0, ln=ln, off=off, tail=tail):
            pltpu.sync_copy(lin_hbm.at[0, pl.ds(r0, ln)], plane.at[pl.ds(0, ln)])
            if tail >= 0:
                pltpu.sync_copy(lintail_hbm, plane.at[pl.ds(tail, 128)])
            pltpu.sync_copy(xt_hbm.at[f, pl.ds(b0, _BSC)], idxbuf)

            def j_body(j, c):
                i16 = idxbuf[pl.ds(j * _L, _L)] + off
                w = plsc.load_gather(plane, [i16])
                plsc.addupdate(m_acc.at[pl.ds(j * _L, _L)], -(w + w))
                return c

            lax.fori_loop(0, _BSC // _L, j_body, 0)

    s_pub.wait()
    plsc.subcore_barrier()
    for d in range(_D):
        pltpu.sync_copy(ssh.at[pl.ds(xbase + d * _BSC + sub, nsub)],
                        s_acc.at[pl.ds(d * nsub, nsub)])

    def s_body(g, c):
        acc = zero
        for d in range(_D):
            sv = s_acc[pl.ds(d * nsub + g * _L, _L)]
            acc = acc + sv * sv
        outbuf[pl.ds(g * _L, _L)] = acc
        return c

    lax.fori_loop(0, nsub // _L, s_body, 0)

    plsc.subcore_barrier()
    pltpu.sync_copy(m_acc, ssh.at[pl.ds(xbase + sid * _BSC, _BSC)])
    plsc.subcore_barrier()
    for d in range(_D):
        pltpu.sync_copy(ssh.at[pl.ds(xbase + d * _BSC + sub, nsub)],
                        s_acc.at[pl.ds(d * nsub, nsub)])

    def m_body(g, c):
        mtot = zero
        for d in range(_D):
            mtot = mtot + s_acc[pl.ds(d * nsub + g * _L, _L)]
        sl = pl.ds(g * _L, _L)
        outbuf[sl] = 0.5 * (outbuf[sl] - mtot)
        return c

    lax.fori_loop(0, nsub // _L, m_body, 0)
    pltpu.sync_copy(outbuf, out_hbm.at[pl.ds(b0 + sub, nsub)])


@jax.jit
def _fm(xt, embT, lin_flat, emb_tail, lin_tail):
    run = functools.partial(
        pl.kernel,
        out_type=jax.ShapeDtypeStruct((_B,), jnp.float32),
        mesh=plsc.VectorSubcoreMesh(core_axis_name="c", subcore_axis_name="s"),
        compiler_params=pltpu.CompilerParams(
            use_tc_tiling_on_sc=True, needs_layout_passes=False,
            disable_bounds_checks=True),
        scratch_types=[
            pltpu.VMEM((_PLANE,), jnp.float32),
            pltpu.VMEM((_BSC,), jnp.float32),
            pltpu.VMEM((_BSC,), jnp.float32),
            pltpu.VMEM((_BSC,), jnp.int32),
            pltpu.VMEM((_BSC // 16,), jnp.float32),
            pltpu.HBM((2 * 16 * _BSC,), jnp.float32),
            pltpu.SemaphoreType.DMA,
            pltpu.SemaphoreType.DMA,
        ],
    )(_fm_body)
    return run(xt, embT, lin_flat, emb_tail, lin_tail)


def kernel(x, emb_table, lin_weight, lin_bias):
    emb_tail = jnp.pad(emb_table[_V - 64:].T, ((0, 0), (0, 64)))
    lin_tail = jnp.pad(lin_weight[_V - 64:, 0], (0, 64))
    out = _fm(x.T, emb_table.T, lin_weight.T, emb_tail, lin_tail)
    return out[:, None] + lin_bias[None, :]
